# Initial kernel scaffold; baseline (speedup 1.0000x reference)
import jax
import jax.numpy as jnp
from jax.experimental import pallas as pl


def kernel(x, edge_index, eps, W1, b1, Wmu, bmu, Wls, bls, Wmu_a, bmu_a, Wls_a, bls_a):
    raise NotImplementedError("write your pallas kernel here")


# speedup vs baseline: 16.5420x; 16.5420x over previous
import functools
import jax
import jax.numpy as jnp
from jax import lax
from jax.experimental import pallas as pl
from jax.experimental.pallas import tpu as pltpu
from jax.experimental.pallas import tpu_sc as plsc

N = 10000
E = 320000
NP = 10240
CH = 125
EROWS = E // CH
D_IN = 128
D_HID = 256
D_LAT = 64
D_ADD = 16
D_OUT = D_LAT + D_ADD
ROWS_PER_TILE = NP // 16
WB = 128

_f32 = jnp.float32
_mesh = plsc.VectorSubcoreMesh(core_axis_name="c", subcore_axis_name="s")


def _deg_body(dst_hbm, out_hbm, acc, dstv, ones_v, wb_v, sem):
  cid = lax.axis_index("c")
  sid = lax.axis_index("s")

  def fill(i, _):
    ones_v[pl.ds(i * 16, 16)] = jnp.full((16,), 1.0, _f32)
    wb_v[pl.ds(i * 16, 16)] = jnp.zeros((16,), _f32)
    return 0
  lax.fori_loop(0, WB // 16, fill, 0)

  def zloop(c, _):
    pltpu.sync_copy(wb_v.at[pl.ds(0, WB)],
                    acc.at[pl.ds(sid * ROWS_PER_TILE + c * WB, WB)])
    return 0
  lax.fori_loop(0, ROWS_PER_TILE // WB, zloop, 0)
  plsc.subcore_barrier()

  nrows = EROWS // 32
  base = cid * (EROWS // 2) + sid * nrows
  pltpu.sync_copy(dst_hbm.at[pl.ds(base, nrows)], dstv)

  def body(j, _):
    pltpu.sync_copy(ones_v.at[pl.ds(0, CH)], acc.at[dstv.at[j]], add=True)
    return 0
  lax.fori_loop(0, nrows, body, 0)
  plsc.subcore_barrier()

  r0 = sid * ROWS_PER_TILE
  def wloop(c, _):
    pltpu.sync_copy(acc.at[pl.ds(r0 + c * WB, WB)], wb_v)
    pltpu.sync_copy(wb_v, out_hbm.at[cid, pl.ds(r0 + c * WB, WB)])
    return 0
  lax.fori_loop(0, ROWS_PER_TILE // WB, wloop, 0)


_deg_call = pl.kernel(
    _deg_body,
    out_type=jax.ShapeDtypeStruct((2, NP), _f32),
    mesh=_mesh,
    scratch_types=[
        pltpu.VMEM_SHARED((NP,), _f32),
        pltpu.VMEM((EROWS // 32, CH), jnp.int32),
        pltpu.VMEM((WB,), _f32),
        pltpu.VMEM((WB,), _f32),
        pltpu.SemaphoreType.DMA,
    ],
)


_IB = 16
_ZB = 64


def _spmm_body(fh, src_hbm, dst_hbm, tbl_hbm, out_hbm,
               acc, srcv, dstv, rows_v, zb_v, sem):
  cid = lax.axis_index("c")
  sid = lax.axis_index("s")

  def fill(i, _):
    r = i // (fh // 16)
    c = lax.rem(i, fh // 16)
    zb_v[r, pl.ds(c * 16, 16)] = jnp.zeros((16,), _f32)
    return 0
  lax.fori_loop(0, _ZB * (fh // 16), fill, 0)

  def zloop(c, _):
    pltpu.sync_copy(zb_v, acc.at[pl.ds(sid * ROWS_PER_TILE + c * _ZB, _ZB)])
    return 0
  lax.fori_loop(0, ROWS_PER_TILE // _ZB, zloop, 0)
  plsc.subcore_barrier()

  nrows = EROWS // 16
  base = sid * nrows

  def outer(b, _):
    pltpu.sync_copy(src_hbm.at[pl.ds(base + b * _IB, _IB)], srcv)
    pltpu.sync_copy(dst_hbm.at[pl.ds(base + b * _IB, _IB)], dstv)

    def body(j, _):
      pltpu.async_copy(tbl_hbm.at[cid].at[srcv.at[j]], rows_v, sem).wait()
      pltpu.sync_copy(rows_v, acc.at[dstv.at[j]], add=True)
      return 0
    lax.fori_loop(0, _IB, body, 0)
    return 0
  lax.fori_loop(0, nrows // _IB, outer, 0)
  plsc.subcore_barrier()

  r0 = sid * ROWS_PER_TILE
  def wloop(c, _):
    pltpu.sync_copy(acc.at[pl.ds(r0 + c * _ZB, _ZB)], zb_v)
    pltpu.sync_copy(zb_v, out_hbm.at[cid, pl.ds(r0 + c * _ZB, _ZB)])
    return 0
  lax.fori_loop(0, ROWS_PER_TILE // _ZB, wloop, 0)


def _make_spmm(fh):
  return pl.kernel(
      functools.partial(_spmm_body, fh),
      out_type=jax.ShapeDtypeStruct((2, NP, fh), _f32),
      mesh=_mesh,
      scratch_types=[
          pltpu.VMEM_SHARED((NP, fh), _f32),
          pltpu.VMEM((_IB, CH), jnp.int32),
          pltpu.VMEM((_IB, CH), jnp.int32),
          pltpu.VMEM((CH, fh), _f32),
          pltpu.VMEM((_ZB, fh), _f32),
          pltpu.SemaphoreType.DMA,
      ],
  )


_spmm_call = _make_spmm(D_HID // 2)


_RB = 1024


def _enc_body(x_ref, dega_ref, degb_ref, w1_ref, g1_ref, dis_ref):
  deg = dega_ref[...] + degb_ref[...] + 1.0
  dis = lax.rsqrt(jnp.maximum(deg, 1.0))
  u = jnp.dot(jnp.log1p(x_ref[...]), w1_ref[...],
              preferred_element_type=_f32)
  g = dis * u
  g1_ref[0] = g[:, :D_HID // 2]
  g1_ref[1] = g[:, D_HID // 2:]
  dis_ref[...] = dis


def _enc_call(x_p, dega, degb, w1):
  return pl.pallas_call(
      _enc_body,
      grid=(NP // _RB,),
      in_specs=[
          pl.BlockSpec((_RB, D_IN), lambda i: (i, 0)),
          pl.BlockSpec((_RB, 1), lambda i: (i, 0)),
          pl.BlockSpec((_RB, 1), lambda i: (i, 0)),
          pl.BlockSpec((D_IN, D_HID), lambda i: (0, 0)),
      ],
      out_specs=[
          pl.BlockSpec((2, _RB, D_HID // 2), lambda i: (0, i, 0)),
          pl.BlockSpec((_RB, 1), lambda i: (i, 0)),
      ],
      out_shape=[
          jax.ShapeDtypeStruct((2, NP, D_HID // 2), _f32),
          jax.ShapeDtypeStruct((NP, 1), _f32),
      ],
  )(x_p, dega, degb, w1)


def _mid_body(acc_ref, g1_ref, dis_ref, b1_ref, wall_ref, g2_ref):
  pre_a = acc_ref[0] + g1_ref[0]
  pre_b = acc_ref[1] + g1_ref[1]
  pre = jnp.concatenate([pre_a, pre_b], axis=1)
  hidden = jnp.maximum(dis_ref[...] * pre + b1_ref[...], 0.0)
  g2 = dis_ref[...] * jnp.dot(hidden, wall_ref[...],
                              preferred_element_type=_f32)
  zpad = jnp.zeros((g2.shape[0], 128 - D_OUT), _f32)
  g2_ref[0] = jnp.concatenate([g2[:, :D_OUT], zpad], axis=1)
  g2_ref[1] = jnp.concatenate([g2[:, D_OUT:], zpad], axis=1)


def _mid_call(acc1, g1, dis, b1, wall):
  return pl.pallas_call(
      _mid_body,
      grid=(NP // _RB,),
      in_specs=[
          pl.BlockSpec((2, _RB, D_HID // 2), lambda i: (0, i, 0)),
          pl.BlockSpec((2, _RB, D_HID // 2), lambda i: (0, i, 0)),
          pl.BlockSpec((_RB, 1), lambda i: (i, 0)),
          pl.BlockSpec((1, D_HID), lambda i: (0, 0)),
          pl.BlockSpec((D_HID, 2 * D_OUT), lambda i: (0, 0)),
      ],
      out_specs=pl.BlockSpec((2, _RB, 128), lambda i: (0, i, 0)),
      out_shape=jax.ShapeDtypeStruct((2, NP, 128), _f32),
  )(acc1, g1, dis, b1, wall)


def _fin_body(acc_ref, g2_ref, dis_ref, bmu_ref, bls_ref, eps_ref,
              mu_ref, ls_ref, z_ref):
  dis = dis_ref[...]
  mu = dis * (acc_ref[0][:, :D_OUT] + g2_ref[0][:, :D_OUT]) + bmu_ref[...]
  ls = dis * (acc_ref[1][:, :D_OUT] + g2_ref[1][:, :D_OUT]) + bls_ref[...]
  z = mu + eps_ref[...] * jnp.exp(ls)
  mu_ref[...] = mu
  ls_ref[...] = ls
  z_ref[...] = jnp.concatenate(
      [z, jnp.zeros((z.shape[0], 128 - D_OUT), _f32)], axis=1)


def _fin_call(acc2, g2, dis, bmu2, bls2, eps_p):
  return pl.pallas_call(
      _fin_body,
      grid=(NP // _RB,),
      in_specs=[
          pl.BlockSpec((2, _RB, 128), lambda i: (0, i, 0)),
          pl.BlockSpec((2, _RB, 128), lambda i: (0, i, 0)),
          pl.BlockSpec((_RB, 1), lambda i: (i, 0)),
          pl.BlockSpec((1, D_OUT), lambda i: (0, 0)),
          pl.BlockSpec((1, D_OUT), lambda i: (0, 0)),
          pl.BlockSpec((_RB, D_OUT), lambda i: (i, 0)),
      ],
      out_specs=[
          pl.BlockSpec((_RB, D_OUT), lambda i: (i, 0)),
          pl.BlockSpec((_RB, D_OUT), lambda i: (i, 0)),
          pl.BlockSpec((_RB, 128), lambda i: (i, 0)),
      ],
      out_shape=[
          jax.ShapeDtypeStruct((NP, D_OUT), _f32),
          jax.ShapeDtypeStruct((NP, D_OUT), _f32),
          jax.ShapeDtypeStruct((NP, 128), _f32),
      ],
  )(acc2, g2, dis, bmu2, bls2, eps_p)


_RA = 400


def _adj_body(zr_ref, zc_ref, out_ref):
  out_ref[...] = lax.dot_general(
      zr_ref[...], zc_ref[...], (((1,), (1,)), ((), ())),
      preferred_element_type=_f32)


def _adj_call(z_p, z_n):
  return pl.pallas_call(
      _adj_body,
      grid=(N // _RA,),
      in_specs=[
          pl.BlockSpec((_RA, 128), lambda i: (i, 0)),
          pl.BlockSpec((N, 128), lambda i: (0, 0)),
      ],
      out_specs=pl.BlockSpec((_RA, N), lambda i: (i, 0)),
      out_shape=jax.ShapeDtypeStruct((N, N), _f32),
  )(z_p, z_n)


def kernel(x, edge_index, eps, W1, b1, Wmu, bmu, Wls, bls,
           Wmu_a, bmu_a, Wls_a, bls_a):
  src2d = edge_index[0].reshape(EROWS, CH)
  dst2d = edge_index[1].reshape(EROWS, CH)
  x_p = jnp.pad(x, ((0, NP - N), (0, 0)))
  eps_p = jnp.pad(eps, ((0, NP - N), (0, 0)))
  wall = jnp.concatenate([Wmu, Wmu_a, Wls, Wls_a], axis=1)
  bmu2 = jnp.concatenate([bmu, bmu_a]).reshape(1, D_OUT)
  bls2 = jnp.concatenate([bls, bls_a]).reshape(1, D_OUT)

  deg2 = _deg_call(dst2d)
  dega = deg2[0].reshape(NP, 1)
  degb = deg2[1].reshape(NP, 1)

  g1, dis = _enc_call(x_p, dega, degb, W1)
  acc1 = _spmm_call(src2d, dst2d, g1)
  g2 = _mid_call(acc1, g1, dis, b1.reshape(1, D_HID), wall)
  acc2 = _spmm_call(src2d, dst2d, g2)
  mu_p, ls_p, z_p = _fin_call(acc2, g2, dis, bmu2, bls2, eps_p)

  adj = _adj_call(z_p, z_p[:N])
  return adj, mu_p[:N], ls_p[:N]


# speedup vs baseline: 22.4312x; 1.3560x over previous
import functools
import jax
import jax.numpy as jnp
from jax import lax
from jax.experimental import pallas as pl
from jax.experimental.pallas import tpu as pltpu
from jax.experimental.pallas import tpu_sc as plsc

N = 10000
E = 320000
NP = 10240
CH = 125
EROWS = E // CH
D_IN = 128
D_HID = 256
D_LAT = 64
D_ADD = 16
D_OUT = D_LAT + D_ADD
ROWS_PER_TILE = NP // 16
WB = 128

_f32 = jnp.float32
_mesh = plsc.VectorSubcoreMesh(core_axis_name="c", subcore_axis_name="s")


def _deg_body(dst_hbm, out_hbm, acc, dstv, ones_v, wb_v, sem):
  cid = lax.axis_index("c")
  sid = lax.axis_index("s")

  def fill(i, _):
    ones_v[pl.ds(i * 16, 16)] = jnp.full((16,), 1.0, _f32)
    wb_v[pl.ds(i * 16, 16)] = jnp.zeros((16,), _f32)
    return 0
  lax.fori_loop(0, WB // 16, fill, 0)

  def zloop(c, _):
    pltpu.sync_copy(wb_v.at[pl.ds(0, WB)],
                    acc.at[pl.ds(sid * ROWS_PER_TILE + c * WB, WB)])
    return 0
  lax.fori_loop(0, ROWS_PER_TILE // WB, zloop, 0)
  plsc.subcore_barrier()

  nrows = EROWS // 32
  base = cid * (EROWS // 2) + sid * nrows
  pltpu.sync_copy(dst_hbm.at[pl.ds(base, nrows)], dstv)

  def body(j, _):
    pltpu.sync_copy(ones_v.at[pl.ds(0, CH)], acc.at[dstv.at[j]], add=True)
    return 0
  lax.fori_loop(0, nrows, body, 0)
  plsc.subcore_barrier()

  r0 = sid * ROWS_PER_TILE
  def wloop(c, _):
    pltpu.sync_copy(acc.at[pl.ds(r0 + c * WB, WB)], wb_v)
    pltpu.sync_copy(wb_v, out_hbm.at[cid, pl.ds(r0 + c * WB, WB)])
    return 0
  lax.fori_loop(0, ROWS_PER_TILE // WB, wloop, 0)


_deg_call = pl.kernel(
    _deg_body,
    out_type=jax.ShapeDtypeStruct((2, NP), _f32),
    mesh=_mesh,
    scratch_types=[
        pltpu.VMEM_SHARED((NP,), _f32),
        pltpu.VMEM((EROWS // 32, CH), jnp.int32),
        pltpu.VMEM((WB,), _f32),
        pltpu.VMEM((WB,), _f32),
        pltpu.SemaphoreType.DMA,
    ],
)


_IB = 16
_ZB = 32


def _spmm_body(fh, src_hbm, dst_hbm, tbl_hbm, out_hbm,
               acc, srcv, dstv, rows0_v, rows1_v, zb_v, sem0, sem1):
  cid = lax.axis_index("c")
  sid = lax.axis_index("s")

  def fill(i, _):
    r = i // (fh // 16)
    c = lax.rem(i, fh // 16)
    zb_v[r, pl.ds(c * 16, 16)] = jnp.zeros((16,), _f32)
    return 0
  lax.fori_loop(0, _ZB * (fh // 16), fill, 0)

  def zloop(c, _):
    pltpu.sync_copy(zb_v, acc.at[pl.ds(sid * ROWS_PER_TILE + c * _ZB, _ZB)])
    return 0
  lax.fori_loop(0, ROWS_PER_TILE // _ZB, zloop, 0)
  plsc.subcore_barrier()

  nrows = EROWS // 16
  base = sid * nrows
  bufs = (rows0_v, rows1_v)
  sems = (sem0, sem1)

  def outer(b, _):
    pltpu.sync_copy(src_hbm.at[pl.ds(base + b * _IB, _IB)], srcv)
    pltpu.sync_copy(dst_hbm.at[pl.ds(base + b * _IB, _IB)], dstv)
    gathers = [None, None]
    gathers[0] = pltpu.async_copy(tbl_hbm.at[cid].at[srcv.at[0]],
                                  bufs[0], sems[0])
    for j in range(_IB):
      p = j % 2
      if j + 1 < _IB:
        gathers[1 - p] = pltpu.async_copy(
            tbl_hbm.at[cid].at[srcv.at[j + 1]], bufs[1 - p], sems[1 - p])
      gathers[p].wait()
      pltpu.sync_copy(bufs[p], acc.at[dstv.at[j]], add=True)
    return 0
  lax.fori_loop(0, nrows // _IB, outer, 0)
  plsc.subcore_barrier()

  r0 = sid * ROWS_PER_TILE
  def wloop(c, _):
    pltpu.sync_copy(acc.at[pl.ds(r0 + c * _ZB, _ZB)], zb_v)
    pltpu.sync_copy(zb_v, out_hbm.at[cid, pl.ds(r0 + c * _ZB, _ZB)])
    return 0
  lax.fori_loop(0, ROWS_PER_TILE // _ZB, wloop, 0)


def _make_spmm(fh):
  return pl.kernel(
      functools.partial(_spmm_body, fh),
      out_type=jax.ShapeDtypeStruct((2, NP, fh), _f32),
      mesh=_mesh,
      scratch_types=[
          pltpu.VMEM_SHARED((NP, fh), _f32),
          pltpu.VMEM((_IB, CH), jnp.int32),
          pltpu.VMEM((_IB, CH), jnp.int32),
          pltpu.VMEM((CH, fh), _f32),
          pltpu.VMEM((CH, fh), _f32),
          pltpu.VMEM((_ZB, fh), _f32),
          pltpu.SemaphoreType.DMA,
          pltpu.SemaphoreType.DMA,
      ],
  )


_spmm_call = _make_spmm(D_HID // 2)


_RB = 1024


def _enc_body(x_ref, dega_ref, degb_ref, w1_ref, g1_ref, dis_ref):
  deg = dega_ref[...] + degb_ref[...] + 1.0
  dis = lax.rsqrt(jnp.maximum(deg, 1.0))
  u = jnp.dot(jnp.log1p(x_ref[...]), w1_ref[...],
              preferred_element_type=_f32)
  g = dis * u
  g1_ref[0] = g[:, :D_HID // 2]
  g1_ref[1] = g[:, D_HID // 2:]
  dis_ref[...] = dis


def _enc_call(x_p, dega, degb, w1):
  return pl.pallas_call(
      _enc_body,
      grid=(NP // _RB,),
      in_specs=[
          pl.BlockSpec((_RB, D_IN), lambda i: (i, 0)),
          pl.BlockSpec((_RB, 1), lambda i: (i, 0)),
          pl.BlockSpec((_RB, 1), lambda i: (i, 0)),
          pl.BlockSpec((D_IN, D_HID), lambda i: (0, 0)),
      ],
      out_specs=[
          pl.BlockSpec((2, _RB, D_HID // 2), lambda i: (0, i, 0)),
          pl.BlockSpec((_RB, 1), lambda i: (i, 0)),
      ],
      out_shape=[
          jax.ShapeDtypeStruct((2, NP, D_HID // 2), _f32),
          jax.ShapeDtypeStruct((NP, 1), _f32),
      ],
  )(x_p, dega, degb, w1)


def _mid_body(acc_ref, g1_ref, dis_ref, b1_ref, wall_ref, g2_ref):
  pre_a = acc_ref[0] + g1_ref[0]
  pre_b = acc_ref[1] + g1_ref[1]
  pre = jnp.concatenate([pre_a, pre_b], axis=1)
  hidden = jnp.maximum(dis_ref[...] * pre + b1_ref[...], 0.0)
  g2 = dis_ref[...] * jnp.dot(hidden, wall_ref[...],
                              preferred_element_type=_f32)
  zpad = jnp.zeros((g2.shape[0], 128 - D_OUT), _f32)
  g2_ref[0] = jnp.concatenate([g2[:, :D_OUT], zpad], axis=1)
  g2_ref[1] = jnp.concatenate([g2[:, D_OUT:], zpad], axis=1)


def _mid_call(acc1, g1, dis, b1, wall):
  return pl.pallas_call(
      _mid_body,
      grid=(NP // _RB,),
      in_specs=[
          pl.BlockSpec((2, _RB, D_HID // 2), lambda i: (0, i, 0)),
          pl.BlockSpec((2, _RB, D_HID // 2), lambda i: (0, i, 0)),
          pl.BlockSpec((_RB, 1), lambda i: (i, 0)),
          pl.BlockSpec((1, D_HID), lambda i: (0, 0)),
          pl.BlockSpec((D_HID, 2 * D_OUT), lambda i: (0, 0)),
      ],
      out_specs=pl.BlockSpec((2, _RB, 128), lambda i: (0, i, 0)),
      out_shape=jax.ShapeDtypeStruct((2, NP, 128), _f32),
  )(acc1, g1, dis, b1, wall)


def _fin_body(acc_ref, g2_ref, dis_ref, bmu_ref, bls_ref, eps_ref,
              mu_ref, ls_ref, z_ref):
  dis = dis_ref[...]
  mu = dis * (acc_ref[0][:, :D_OUT] + g2_ref[0][:, :D_OUT]) + bmu_ref[...]
  ls = dis * (acc_ref[1][:, :D_OUT] + g2_ref[1][:, :D_OUT]) + bls_ref[...]
  z = mu + eps_ref[...] * jnp.exp(ls)
  mu_ref[...] = mu
  ls_ref[...] = ls
  z_ref[...] = jnp.concatenate(
      [z, jnp.zeros((z.shape[0], 128 - D_OUT), _f32)], axis=1)


def _fin_call(acc2, g2, dis, bmu2, bls2, eps_p):
  return pl.pallas_call(
      _fin_body,
      grid=(NP // _RB,),
      in_specs=[
          pl.BlockSpec((2, _RB, 128), lambda i: (0, i, 0)),
          pl.BlockSpec((2, _RB, 128), lambda i: (0, i, 0)),
          pl.BlockSpec((_RB, 1), lambda i: (i, 0)),
          pl.BlockSpec((1, D_OUT), lambda i: (0, 0)),
          pl.BlockSpec((1, D_OUT), lambda i: (0, 0)),
          pl.BlockSpec((_RB, D_OUT), lambda i: (i, 0)),
      ],
      out_specs=[
          pl.BlockSpec((_RB, D_OUT), lambda i: (i, 0)),
          pl.BlockSpec((_RB, D_OUT), lambda i: (i, 0)),
          pl.BlockSpec((_RB, 128), lambda i: (i, 0)),
      ],
      out_shape=[
          jax.ShapeDtypeStruct((NP, D_OUT), _f32),
          jax.ShapeDtypeStruct((NP, D_OUT), _f32),
          jax.ShapeDtypeStruct((NP, 128), _f32),
      ],
  )(acc2, g2, dis, bmu2, bls2, eps_p)


_RA = 400


def _adj_body(zr_ref, zc_ref, out_ref):
  out_ref[...] = lax.dot_general(
      zr_ref[...], zc_ref[...], (((1,), (1,)), ((), ())),
      preferred_element_type=_f32)


def _adj_call(z_p, z_n):
  return pl.pallas_call(
      _adj_body,
      grid=(N // _RA,),
      in_specs=[
          pl.BlockSpec((_RA, 128), lambda i: (i, 0)),
          pl.BlockSpec((N, 128), lambda i: (0, 0)),
      ],
      out_specs=pl.BlockSpec((_RA, N), lambda i: (i, 0)),
      out_shape=jax.ShapeDtypeStruct((N, N), _f32),
  )(z_p, z_n)


def kernel(x, edge_index, eps, W1, b1, Wmu, bmu, Wls, bls,
           Wmu_a, bmu_a, Wls_a, bls_a):
  src2d = edge_index[0].reshape(EROWS, CH)
  dst2d = edge_index[1].reshape(EROWS, CH)
  x_p = jnp.pad(x, ((0, NP - N), (0, 0)))
  eps_p = jnp.pad(eps, ((0, NP - N), (0, 0)))
  wall = jnp.concatenate([Wmu, Wmu_a, Wls, Wls_a], axis=1)
  bmu2 = jnp.concatenate([bmu, bmu_a]).reshape(1, D_OUT)
  bls2 = jnp.concatenate([bls, bls_a]).reshape(1, D_OUT)

  deg2 = _deg_call(dst2d)
  dega = deg2[0].reshape(NP, 1)
  degb = deg2[1].reshape(NP, 1)

  g1, dis = _enc_call(x_p, dega, degb, W1)
  acc1 = _spmm_call(src2d, dst2d, g1)
  g2 = _mid_call(acc1, g1, dis, b1.reshape(1, D_HID), wall)
  acc2 = _spmm_call(src2d, dst2d, g2)
  mu_p, ls_p, z_p = _fin_call(acc2, g2, dis, bmu2, bls2, eps_p)

  adj = _adj_call(z_p, z_p[:N])
  return adj, mu_p[:N], ls_p[:N]


# speedup vs baseline: 23.6588x; 1.0547x over previous
import functools
import jax
import jax.numpy as jnp
from jax import lax
from jax.experimental import pallas as pl
from jax.experimental.pallas import tpu as pltpu
from jax.experimental.pallas import tpu_sc as plsc

N = 10000
E = 320000
NP = 10240
CH = 125
EROWS = E // CH
D_IN = 128
D_HID = 256
D_LAT = 64
D_ADD = 16
D_OUT = D_LAT + D_ADD
ROWS_PER_TILE = NP // 16
WB = 128

_f32 = jnp.float32
_mesh = plsc.VectorSubcoreMesh(core_axis_name="c", subcore_axis_name="s")


def _deg_body(dst_hbm, out_hbm, acc, dstv, ones_v, wb_v, sem):
  cid = lax.axis_index("c")
  sid = lax.axis_index("s")

  def fill(i, _):
    ones_v[pl.ds(i * 16, 16)] = jnp.full((16,), 1.0, _f32)
    wb_v[pl.ds(i * 16, 16)] = jnp.zeros((16,), _f32)
    return 0
  lax.fori_loop(0, WB // 16, fill, 0)

  def zloop(c, _):
    pltpu.sync_copy(wb_v.at[pl.ds(0, WB)],
                    acc.at[pl.ds(sid * ROWS_PER_TILE + c * WB, WB)])
    return 0
  lax.fori_loop(0, ROWS_PER_TILE // WB, zloop, 0)
  plsc.subcore_barrier()

  nrows = EROWS // 32
  base = cid * (EROWS // 2) + sid * nrows
  pltpu.sync_copy(dst_hbm.at[pl.ds(base, nrows)], dstv)

  def body(j, _):
    pltpu.sync_copy(ones_v.at[pl.ds(0, CH)], acc.at[dstv.at[j]], add=True)
    return 0
  lax.fori_loop(0, nrows, body, 0)
  plsc.subcore_barrier()

  r0 = sid * ROWS_PER_TILE
  def wloop(c, _):
    pltpu.sync_copy(acc.at[pl.ds(r0 + c * WB, WB)], wb_v)
    pltpu.sync_copy(wb_v, out_hbm.at[cid, pl.ds(r0 + c * WB, WB)])
    return 0
  lax.fori_loop(0, ROWS_PER_TILE // WB, wloop, 0)


_deg_call = pl.kernel(
    _deg_body,
    out_type=jax.ShapeDtypeStruct((2, NP), _f32),
    mesh=_mesh,
    scratch_types=[
        pltpu.VMEM_SHARED((NP,), _f32),
        pltpu.VMEM((EROWS // 32, CH), jnp.int32),
        pltpu.VMEM((WB,), _f32),
        pltpu.VMEM((WB,), _f32),
        pltpu.SemaphoreType.DMA,
    ],
)


_IB = 16
_ZB = 32


def _spmm_body(fh, src_hbm, dst_hbm, tbl_hbm, out_hbm,
               acc, srcv, dstv, rows0_v, rows1_v, zb_v, sem0, sem1):
  cid = lax.axis_index("c")
  sid = lax.axis_index("s")

  def fill(i, _):
    r = i // (fh // 16)
    c = lax.rem(i, fh // 16)
    zb_v[r, pl.ds(c * 16, 16)] = jnp.zeros((16,), _f32)
    return 0
  lax.fori_loop(0, _ZB * (fh // 16), fill, 0)

  def zloop(c, _):
    pltpu.sync_copy(zb_v, acc.at[pl.ds(sid * ROWS_PER_TILE + c * _ZB, _ZB)])
    return 0
  lax.fori_loop(0, ROWS_PER_TILE // _ZB, zloop, 0)
  plsc.subcore_barrier()

  nrows = EROWS // 16
  base = sid * nrows
  bufs = (rows0_v, rows1_v)
  sems = (sem0, sem1)

  def outer(b, _):
    pltpu.sync_copy(src_hbm.at[pl.ds(base + b * _IB, _IB)], srcv)
    pltpu.sync_copy(dst_hbm.at[pl.ds(base + b * _IB, _IB)], dstv)
    gathers = [None, None]
    gathers[0] = pltpu.async_copy(tbl_hbm.at[cid].at[srcv.at[0]],
                                  bufs[0], sems[0])
    for j in range(_IB):
      p = j % 2
      if j + 1 < _IB:
        gathers[1 - p] = pltpu.async_copy(
            tbl_hbm.at[cid].at[srcv.at[j + 1]], bufs[1 - p], sems[1 - p])
      gathers[p].wait()
      pltpu.sync_copy(bufs[p], acc.at[dstv.at[j]], add=True)
    return 0
  lax.fori_loop(0, nrows // _IB, outer, 0)
  plsc.subcore_barrier()

  r0 = sid * ROWS_PER_TILE
  def wloop(c, _):
    pltpu.sync_copy(acc.at[pl.ds(r0 + c * _ZB, _ZB)], zb_v)
    pltpu.sync_copy(zb_v, out_hbm.at[cid, pl.ds(r0 + c * _ZB, _ZB)])
    return 0
  lax.fori_loop(0, ROWS_PER_TILE // _ZB, wloop, 0)


def _make_spmm(fh, **kernel_kwargs):
  return pl.kernel(
      functools.partial(_spmm_body, fh),
      out_type=jax.ShapeDtypeStruct((2, NP, fh), _f32),
      mesh=_mesh,
      scratch_types=[
          pltpu.VMEM_SHARED((NP, fh), _f32),
          pltpu.VMEM((_IB, CH), jnp.int32),
          pltpu.VMEM((_IB, CH), jnp.int32),
          pltpu.VMEM((CH, fh), _f32),
          pltpu.VMEM((CH, fh), _f32),
          pltpu.VMEM((_ZB, fh), _f32),
          pltpu.SemaphoreType.DMA,
          pltpu.SemaphoreType.DMA,
      ],
      **kernel_kwargs,
  )


_spmm_call = _make_spmm(D_HID // 2)
_spmm2_call = _make_spmm(
    D_OUT,
    compiler_params=pltpu.CompilerParams(use_tc_tiling_on_sc=False))


_RB = 1024


def _enc_body(x_ref, dega_ref, degb_ref, w1_ref, g1_ref, dis_ref):
  deg = dega_ref[...] + degb_ref[...] + 1.0
  dis = lax.rsqrt(jnp.maximum(deg, 1.0))
  u = jnp.dot(jnp.log1p(x_ref[...]), w1_ref[...],
              preferred_element_type=_f32)
  g = dis * u
  g1_ref[0] = g[:, :D_HID // 2]
  g1_ref[1] = g[:, D_HID // 2:]
  dis_ref[...] = dis


def _enc_call(x_p, dega, degb, w1):
  return pl.pallas_call(
      _enc_body,
      grid=(NP // _RB,),
      in_specs=[
          pl.BlockSpec((_RB, D_IN), lambda i: (i, 0)),
          pl.BlockSpec((_RB, 1), lambda i: (i, 0)),
          pl.BlockSpec((_RB, 1), lambda i: (i, 0)),
          pl.BlockSpec((D_IN, D_HID), lambda i: (0, 0)),
      ],
      out_specs=[
          pl.BlockSpec((2, _RB, D_HID // 2), lambda i: (0, i, 0)),
          pl.BlockSpec((_RB, 1), lambda i: (i, 0)),
      ],
      out_shape=[
          jax.ShapeDtypeStruct((2, NP, D_HID // 2), _f32),
          jax.ShapeDtypeStruct((NP, 1), _f32),
      ],
  )(x_p, dega, degb, w1)


def _mid_body(acc_ref, g1_ref, dis_ref, b1_ref, wall_ref, g2_ref):
  pre_a = acc_ref[0] + g1_ref[0]
  pre_b = acc_ref[1] + g1_ref[1]
  pre = jnp.concatenate([pre_a, pre_b], axis=1)
  hidden = jnp.maximum(dis_ref[...] * pre + b1_ref[...], 0.0)
  g2 = dis_ref[...] * jnp.dot(hidden, wall_ref[...],
                              preferred_element_type=_f32)
  g2_ref[0] = g2[:, :D_OUT]
  g2_ref[1] = g2[:, D_OUT:]


def _mid_call(acc1, g1, dis, b1, wall):
  return pl.pallas_call(
      _mid_body,
      grid=(NP // _RB,),
      in_specs=[
          pl.BlockSpec((2, _RB, D_HID // 2), lambda i: (0, i, 0)),
          pl.BlockSpec((2, _RB, D_HID // 2), lambda i: (0, i, 0)),
          pl.BlockSpec((_RB, 1), lambda i: (i, 0)),
          pl.BlockSpec((1, D_HID), lambda i: (0, 0)),
          pl.BlockSpec((D_HID, 2 * D_OUT), lambda i: (0, 0)),
      ],
      out_specs=pl.BlockSpec((2, _RB, D_OUT), lambda i: (0, i, 0)),
      out_shape=jax.ShapeDtypeStruct((2, NP, D_OUT), _f32),
  )(acc1, g1, dis, b1, wall)


def _fin_body(acc_ref, g2_ref, dis_ref, bmu_ref, bls_ref, eps_ref,
              mu_ref, ls_ref, z_ref):
  dis = dis_ref[...]
  mu = dis * (acc_ref[0] + g2_ref[0]) + bmu_ref[...]
  ls = dis * (acc_ref[1] + g2_ref[1]) + bls_ref[...]
  z = mu + eps_ref[...] * jnp.exp(ls)
  mu_ref[...] = mu
  ls_ref[...] = ls
  z_ref[...] = jnp.concatenate(
      [z, jnp.zeros((z.shape[0], 128 - D_OUT), _f32)], axis=1)


def _fin_call(acc2, g2, dis, bmu2, bls2, eps_p):
  return pl.pallas_call(
      _fin_body,
      grid=(NP // _RB,),
      in_specs=[
          pl.BlockSpec((2, _RB, D_OUT), lambda i: (0, i, 0)),
          pl.BlockSpec((2, _RB, D_OUT), lambda i: (0, i, 0)),
          pl.BlockSpec((_RB, 1), lambda i: (i, 0)),
          pl.BlockSpec((1, D_OUT), lambda i: (0, 0)),
          pl.BlockSpec((1, D_OUT), lambda i: (0, 0)),
          pl.BlockSpec((_RB, D_OUT), lambda i: (i, 0)),
      ],
      out_specs=[
          pl.BlockSpec((_RB, D_OUT), lambda i: (i, 0)),
          pl.BlockSpec((_RB, D_OUT), lambda i: (i, 0)),
          pl.BlockSpec((_RB, 128), lambda i: (i, 0)),
      ],
      out_shape=[
          jax.ShapeDtypeStruct((NP, D_OUT), _f32),
          jax.ShapeDtypeStruct((NP, D_OUT), _f32),
          jax.ShapeDtypeStruct((NP, 128), _f32),
      ],
  )(acc2, g2, dis, bmu2, bls2, eps_p)


_RA = 400


def _adj_body(zr_ref, zc_ref, out_ref):
  out_ref[...] = lax.dot_general(
      zr_ref[...], zc_ref[...], (((1,), (1,)), ((), ())),
      preferred_element_type=_f32)


def _adj_call(z_p, z_n):
  return pl.pallas_call(
      _adj_body,
      grid=(N // _RA,),
      in_specs=[
          pl.BlockSpec((_RA, 128), lambda i: (i, 0)),
          pl.BlockSpec((N, 128), lambda i: (0, 0)),
      ],
      out_specs=pl.BlockSpec((_RA, N), lambda i: (i, 0)),
      out_shape=jax.ShapeDtypeStruct((N, N), _f32),
  )(z_p, z_n)


def kernel(x, edge_index, eps, W1, b1, Wmu, bmu, Wls, bls,
           Wmu_a, bmu_a, Wls_a, bls_a):
  src2d = edge_index[0].reshape(EROWS, CH)
  dst2d = edge_index[1].reshape(EROWS, CH)
  x_p = jnp.pad(x, ((0, NP - N), (0, 0)))
  eps_p = jnp.pad(eps, ((0, NP - N), (0, 0)))
  wall = jnp.concatenate([Wmu, Wmu_a, Wls, Wls_a], axis=1)
  bmu2 = jnp.concatenate([bmu, bmu_a]).reshape(1, D_OUT)
  bls2 = jnp.concatenate([bls, bls_a]).reshape(1, D_OUT)

  deg2 = _deg_call(dst2d)
  dega = deg2[0].reshape(NP, 1)
  degb = deg2[1].reshape(NP, 1)

  g1, dis = _enc_call(x_p, dega, degb, W1)
  acc1 = _spmm_call(src2d, dst2d, g1)
  g2 = _mid_call(acc1, g1, dis, b1.reshape(1, D_HID), wall)
  acc2 = _spmm2_call(src2d, dst2d, g2)
  mu_p, ls_p, z_p = _fin_call(acc2, g2, dis, bmu2, bls2, eps_p)

  adj = _adj_call(z_p, z_p[:N])
  return adj, mu_p[:N], ls_p[:N]


# speedup vs baseline: 28.1304x; 1.1890x over previous
import functools
import jax
import jax.numpy as jnp
from jax import lax
from jax.experimental import pallas as pl
from jax.experimental.pallas import tpu as pltpu
from jax.experimental.pallas import tpu_sc as plsc

N = 10000
E = 320000
NP = 10240
CH = 125
EROWS = E // CH
D_IN = 128
D_HID = 256
D_LAT = 64
D_ADD = 16
D_OUT = D_LAT + D_ADD
ROWS_PER_TILE = NP // 16
WB = 128

_f32 = jnp.float32
_mesh = plsc.VectorSubcoreMesh(core_axis_name="c", subcore_axis_name="s")


def _deg_body(dst_hbm, out_hbm, acc, dstv, ones_v, wb_v, sem):
  cid = lax.axis_index("c")
  sid = lax.axis_index("s")

  def fill(i, _):
    ones_v[pl.ds(i * 16, 16)] = jnp.full((16,), 1.0, _f32)
    wb_v[pl.ds(i * 16, 16)] = jnp.zeros((16,), _f32)
    return 0
  lax.fori_loop(0, WB // 16, fill, 0)

  def zloop(c, _):
    pltpu.sync_copy(wb_v.at[pl.ds(0, WB)],
                    acc.at[pl.ds(sid * ROWS_PER_TILE + c * WB, WB)])
    return 0
  lax.fori_loop(0, ROWS_PER_TILE // WB, zloop, 0)
  plsc.subcore_barrier()

  nrows = EROWS // 32
  base = cid * (EROWS // 2) + sid * nrows
  pltpu.sync_copy(dst_hbm.at[pl.ds(base, nrows)], dstv)

  def body(j, _):
    pltpu.sync_copy(ones_v.at[pl.ds(0, CH)], acc.at[dstv.at[j]], add=True)
    return 0
  lax.fori_loop(0, nrows, body, 0)
  plsc.subcore_barrier()

  r0 = sid * ROWS_PER_TILE
  def wloop(c, _):
    pltpu.sync_copy(acc.at[pl.ds(r0 + c * WB, WB)], wb_v)
    pltpu.sync_copy(wb_v, out_hbm.at[cid, pl.ds(r0 + c * WB, WB)])
    return 0
  lax.fori_loop(0, ROWS_PER_TILE // WB, wloop, 0)


_deg_call = pl.kernel(
    _deg_body,
    out_type=jax.ShapeDtypeStruct((2, NP), _f32),
    mesh=_mesh,
    scratch_types=[
        pltpu.VMEM_SHARED((NP,), _f32),
        pltpu.VMEM((EROWS // 32, CH), jnp.int32),
        pltpu.VMEM((WB,), _f32),
        pltpu.VMEM((WB,), _f32),
        pltpu.SemaphoreType.DMA,
    ],
)


_IB = 16
_ZB = 32


def _spmm_body(fh, src_hbm, dst_hbm, tbl_hbm, out_hbm,
               acc, srcv, dstv, rows0_v, rows1_v, zb_v, sem0, sem1):
  cid = lax.axis_index("c")
  sid = lax.axis_index("s")

  def fill(i, _):
    r = i // (fh // 16)
    c = lax.rem(i, fh // 16)
    zb_v[r, pl.ds(c * 16, 16)] = jnp.zeros((16,), _f32)
    return 0
  lax.fori_loop(0, _ZB * (fh // 16), fill, 0)

  def zloop(c, _):
    pltpu.sync_copy(zb_v, acc.at[pl.ds(sid * ROWS_PER_TILE + c * _ZB, _ZB)])
    return 0
  lax.fori_loop(0, ROWS_PER_TILE // _ZB, zloop, 0)
  plsc.subcore_barrier()

  nrows = EROWS // 16
  base = sid * nrows
  bufs = (rows0_v, rows1_v)
  sems = (sem0, sem1)

  def outer(b, _):
    pltpu.sync_copy(src_hbm.at[pl.ds(base + b * _IB, _IB)], srcv)
    pltpu.sync_copy(dst_hbm.at[pl.ds(base + b * _IB, _IB)], dstv)
    gathers = [None, None]
    gathers[0] = pltpu.async_copy(tbl_hbm.at[cid].at[srcv.at[0]],
                                  bufs[0], sems[0])
    for j in range(_IB):
      p = j % 2
      if j + 1 < _IB:
        gathers[1 - p] = pltpu.async_copy(
            tbl_hbm.at[cid].at[srcv.at[j + 1]], bufs[1 - p], sems[1 - p])
      gathers[p].wait()
      pltpu.sync_copy(bufs[p], acc.at[dstv.at[j]], add=True)
    return 0
  lax.fori_loop(0, nrows // _IB, outer, 0)
  plsc.subcore_barrier()

  r0 = sid * ROWS_PER_TILE
  def wloop(c, _):
    pltpu.sync_copy(acc.at[pl.ds(r0 + c * _ZB, _ZB)], zb_v)
    pltpu.sync_copy(zb_v, out_hbm.at[cid, pl.ds(r0 + c * _ZB, _ZB)])
    return 0
  lax.fori_loop(0, ROWS_PER_TILE // _ZB, wloop, 0)


def _make_spmm(fh, **kernel_kwargs):
  return pl.kernel(
      functools.partial(_spmm_body, fh),
      out_type=jax.ShapeDtypeStruct((2, NP, fh), _f32),
      mesh=_mesh,
      scratch_types=[
          pltpu.VMEM_SHARED((NP, fh), _f32),
          pltpu.VMEM((_IB, CH), jnp.int32),
          pltpu.VMEM((_IB, CH), jnp.int32),
          pltpu.VMEM((CH, fh), _f32),
          pltpu.VMEM((CH, fh), _f32),
          pltpu.VMEM((_ZB, fh), _f32),
          pltpu.SemaphoreType.DMA,
          pltpu.SemaphoreType.DMA,
      ],
      **kernel_kwargs,
  )


def _spmm_es_body(src_hbm, dst_hbm, tbl_hbm, out_hbm,
                  acc, srcv, dstv, rows0_v, rows1_v, zb_v, sem0, sem1):
  fh = D_IN
  cid = lax.axis_index("c")
  sid = lax.axis_index("s")

  def fill(i, _):
    r = i // (fh // 16)
    c = lax.rem(i, fh // 16)
    zb_v[r, pl.ds(c * 16, 16)] = jnp.zeros((16,), _f32)
    return 0
  lax.fori_loop(0, _ZB * (fh // 16), fill, 0)

  def zloop(c, _):
    pltpu.sync_copy(zb_v, acc.at[pl.ds(sid * ROWS_PER_TILE + c * _ZB, _ZB)])
    return 0
  lax.fori_loop(0, ROWS_PER_TILE // _ZB, zloop, 0)
  plsc.subcore_barrier()

  nrows = EROWS // 32
  base = cid * (EROWS // 2) + sid * nrows
  bufs = (rows0_v, rows1_v)
  sems = (sem0, sem1)

  def outer(b, _):
    pltpu.sync_copy(src_hbm.at[pl.ds(base + b * _IB, _IB)], srcv)
    pltpu.sync_copy(dst_hbm.at[pl.ds(base + b * _IB, _IB)], dstv)
    gathers = [None, None]
    gathers[0] = pltpu.async_copy(tbl_hbm.at[srcv.at[0]], bufs[0], sems[0])
    for j in range(_IB):
      p = j % 2
      if j + 1 < _IB:
        gathers[1 - p] = pltpu.async_copy(
            tbl_hbm.at[srcv.at[j + 1]], bufs[1 - p], sems[1 - p])
      gathers[p].wait()
      pltpu.sync_copy(bufs[p], acc.at[dstv.at[j]], add=True)
    return 0
  lax.fori_loop(0, nrows // _IB, outer, 0)
  plsc.subcore_barrier()

  r0 = sid * ROWS_PER_TILE
  def wloop(c, _):
    pltpu.sync_copy(acc.at[pl.ds(r0 + c * _ZB, _ZB)], zb_v)
    pltpu.sync_copy(zb_v, out_hbm.at[cid, pl.ds(r0 + c * _ZB, _ZB)])
    return 0
  lax.fori_loop(0, ROWS_PER_TILE // _ZB, wloop, 0)


_spmm_es_call = pl.kernel(
    _spmm_es_body,
    out_type=jax.ShapeDtypeStruct((2, NP, D_IN), _f32),
    mesh=_mesh,
    scratch_types=[
        pltpu.VMEM_SHARED((NP, D_IN), _f32),
        pltpu.VMEM((_IB, CH), jnp.int32),
        pltpu.VMEM((_IB, CH), jnp.int32),
        pltpu.VMEM((CH, D_IN), _f32),
        pltpu.VMEM((CH, D_IN), _f32),
        pltpu.VMEM((_ZB, D_IN), _f32),
        pltpu.SemaphoreType.DMA,
        pltpu.SemaphoreType.DMA,
    ],
)
_spmm2_call = _make_spmm(
    D_OUT,
    compiler_params=pltpu.CompilerParams(use_tc_tiling_on_sc=False))


_RB = 1024


def _enc_body(x_ref, dega_ref, degb_ref, gx_ref, dis_ref):
  deg = dega_ref[...] + degb_ref[...] + 1.0
  dis = lax.rsqrt(jnp.maximum(deg, 1.0))
  gx_ref[...] = dis * jnp.log1p(x_ref[...])
  dis_ref[...] = dis


def _enc_call(x_p, dega, degb):
  return pl.pallas_call(
      _enc_body,
      grid=(NP // _RB,),
      in_specs=[
          pl.BlockSpec((_RB, D_IN), lambda i: (i, 0)),
          pl.BlockSpec((_RB, 1), lambda i: (i, 0)),
          pl.BlockSpec((_RB, 1), lambda i: (i, 0)),
      ],
      out_specs=[
          pl.BlockSpec((_RB, D_IN), lambda i: (i, 0)),
          pl.BlockSpec((_RB, 1), lambda i: (i, 0)),
      ],
      out_shape=[
          jax.ShapeDtypeStruct((NP, D_IN), _f32),
          jax.ShapeDtypeStruct((NP, 1), _f32),
      ],
  )(x_p, dega, degb)


def _mid_body(accp_ref, gx_ref, dis_ref, b1_ref, w1_ref, wall_ref, g2_ref):
  pre = accp_ref[0] + accp_ref[1] + gx_ref[...]
  u = jnp.dot(pre, w1_ref[...], preferred_element_type=_f32)
  hidden = jnp.maximum(dis_ref[...] * u + b1_ref[...], 0.0)
  g2 = dis_ref[...] * jnp.dot(hidden, wall_ref[...],
                              preferred_element_type=_f32)
  g2_ref[0] = g2[:, :D_OUT]
  g2_ref[1] = g2[:, D_OUT:]


def _mid_call(accp, gx, dis, b1, w1, wall):
  return pl.pallas_call(
      _mid_body,
      grid=(NP // _RB,),
      in_specs=[
          pl.BlockSpec((2, _RB, D_IN), lambda i: (0, i, 0)),
          pl.BlockSpec((_RB, D_IN), lambda i: (i, 0)),
          pl.BlockSpec((_RB, 1), lambda i: (i, 0)),
          pl.BlockSpec((1, D_HID), lambda i: (0, 0)),
          pl.BlockSpec((D_IN, D_HID), lambda i: (0, 0)),
          pl.BlockSpec((D_HID, 2 * D_OUT), lambda i: (0, 0)),
      ],
      out_specs=pl.BlockSpec((2, _RB, D_OUT), lambda i: (0, i, 0)),
      out_shape=jax.ShapeDtypeStruct((2, NP, D_OUT), _f32),
  )(accp, gx, dis, b1, w1, wall)


def _fin_body(acc_ref, g2_ref, dis_ref, bmu_ref, bls_ref, eps_ref,
              mu_ref, ls_ref, z_ref):
  dis = dis_ref[...]
  mu = dis * (acc_ref[0] + g2_ref[0]) + bmu_ref[...]
  ls = dis * (acc_ref[1] + g2_ref[1]) + bls_ref[...]
  z = mu + eps_ref[...] * jnp.exp(ls)
  mu_ref[...] = mu
  ls_ref[...] = ls
  z_ref[...] = jnp.concatenate(
      [z, jnp.zeros((z.shape[0], 128 - D_OUT), _f32)], axis=1)


def _fin_call(acc2, g2, dis, bmu2, bls2, eps_p):
  return pl.pallas_call(
      _fin_body,
      grid=(NP // _RB,),
      in_specs=[
          pl.BlockSpec((2, _RB, D_OUT), lambda i: (0, i, 0)),
          pl.BlockSpec((2, _RB, D_OUT), lambda i: (0, i, 0)),
          pl.BlockSpec((_RB, 1), lambda i: (i, 0)),
          pl.BlockSpec((1, D_OUT), lambda i: (0, 0)),
          pl.BlockSpec((1, D_OUT), lambda i: (0, 0)),
          pl.BlockSpec((_RB, D_OUT), lambda i: (i, 0)),
      ],
      out_specs=[
          pl.BlockSpec((_RB, D_OUT), lambda i: (i, 0)),
          pl.BlockSpec((_RB, D_OUT), lambda i: (i, 0)),
          pl.BlockSpec((_RB, 128), lambda i: (i, 0)),
      ],
      out_shape=[
          jax.ShapeDtypeStruct((NP, D_OUT), _f32),
          jax.ShapeDtypeStruct((NP, D_OUT), _f32),
          jax.ShapeDtypeStruct((NP, 128), _f32),
      ],
  )(acc2, g2, dis, bmu2, bls2, eps_p)


_RA = 400


def _adj_body(zr_ref, zc_ref, out_ref):
  out_ref[...] = lax.dot_general(
      zr_ref[...], zc_ref[...], (((1,), (1,)), ((), ())),
      preferred_element_type=_f32)


def _adj_call(z_p, z_n):
  return pl.pallas_call(
      _adj_body,
      grid=(N // _RA,),
      in_specs=[
          pl.BlockSpec((_RA, 128), lambda i: (i, 0)),
          pl.BlockSpec((N, 128), lambda i: (0, 0)),
      ],
      out_specs=pl.BlockSpec((_RA, N), lambda i: (i, 0)),
      out_shape=jax.ShapeDtypeStruct((N, N), _f32),
  )(z_p, z_n)


def kernel(x, edge_index, eps, W1, b1, Wmu, bmu, Wls, bls,
           Wmu_a, bmu_a, Wls_a, bls_a):
  src2d = edge_index[0].reshape(EROWS, CH)
  dst2d = edge_index[1].reshape(EROWS, CH)
  x_p = jnp.pad(x, ((0, NP - N), (0, 0)))
  eps_p = jnp.pad(eps, ((0, NP - N), (0, 0)))
  wall = jnp.concatenate([Wmu, Wmu_a, Wls, Wls_a], axis=1)
  bmu2 = jnp.concatenate([bmu, bmu_a]).reshape(1, D_OUT)
  bls2 = jnp.concatenate([bls, bls_a]).reshape(1, D_OUT)

  deg2 = _deg_call(dst2d)
  dega = deg2[0].reshape(NP, 1)
  degb = deg2[1].reshape(NP, 1)

  gx, dis = _enc_call(x_p, dega, degb)
  accp = _spmm_es_call(src2d, dst2d, gx)
  g2 = _mid_call(accp, gx, dis, b1.reshape(1, D_HID), W1, wall)
  acc2 = _spmm2_call(src2d, dst2d, g2)
  mu_p, ls_p, z_p = _fin_call(acc2, g2, dis, bmu2, bls2, eps_p)

  adj = _adj_call(z_p, z_p[:N])
  return adj, mu_p[:N], ls_p[:N]
